# baseline (speedup 1.0000x reference)
import functools

import jax
import jax.numpy as jnp
from jax.experimental import pallas as pl
from jax.experimental.pallas import tpu as pltpu


def _conv_pool_kernel(xp_ref, w_ref, b_ref, maxp_ref, minp_ref, stats_ref,
                      acc_ref, *, KH, KW):
    _, Ho2, Wo2, Cout = maxp_ref.shape
    Ho, Wo = 2 * Ho2, 2 * Wo2
    Cin = xp_ref.shape[3]
    rows = Ho * Wo

    cols = []
    for kh in range(KH):
        for kw in range(KW):
            cols.append(
                xp_ref[:, pl.ds(kh, Ho), pl.ds(kw, Wo), :].reshape(rows, Cin))
    patch = jnp.concatenate(cols, axis=1)

    acc = jnp.dot(patch, w_ref[...],
                  preferred_element_type=jnp.float32)
    acc = acc + b_ref[0]

    ssum = jnp.sum(acc, axis=0, keepdims=True)
    ssq = jnp.sum(acc * acc, axis=0, keepdims=True)
    stats_ref[...] = jnp.concatenate([ssum, ssq], axis=0).reshape(1, 2, Cout)

    acc_ref[...] = acc.reshape(Ho, Wo, Cout)
    mx = None
    mn = None
    for di in range(2):
        for dj in range(2):
            part = acc_ref[pl.ds(di, Ho2, 2), pl.ds(dj, Wo2, 2), :]
            mx = part if mx is None else jnp.maximum(mx, part)
            mn = part if mn is None else jnp.minimum(mn, part)
    maxp_ref[...] = mx.reshape(1, Ho2, Wo2, Cout).astype(maxp_ref.dtype)
    minp_ref[...] = mn.reshape(1, Ho2, Wo2, Cout).astype(minp_ref.dtype)


def _bn_relu_kernel(stats_ref, g_ref, be_ref, maxp_ref, minp_ref, o_ref,
                    *, count, eps):
    _, Ho2, Wo2, Cout = o_ref.shape

    ssum = jnp.sum(stats_ref[:, 0, :], axis=0, keepdims=True)
    ssq = jnp.sum(stats_ref[:, 1, :], axis=0, keepdims=True)
    mean = ssum / count
    var = ssq / count - mean * mean
    inv = jax.lax.rsqrt(var + eps)
    scale = g_ref[...] * inv
    shift = be_ref[...] - mean * scale

    shape = o_ref.shape
    scale4 = jnp.broadcast_to(scale.reshape(1, 1, 1, Cout), shape)
    shift4 = jnp.broadcast_to(shift.reshape(1, 1, 1, Cout), shape)
    sel = jnp.where(scale4 >= 0.0,
                    maxp_ref[...].astype(jnp.float32),
                    minp_ref[...].astype(jnp.float32))
    o_ref[...] = jnp.maximum(sel * scale4 + shift4, 0.0)


def kernel(x, w, b, gamma, beta):
    stride, padding, eps = 1, 1, 1e-5
    N, Cin, H, W = x.shape
    Cout, _, KH, KW = w.shape
    Ho = (H + 2 * padding - KH) // stride + 1
    Wo = (W + 2 * padding - KW) // stride + 1
    assert Ho % 2 == 0 and Wo % 2 == 0
    Ho2, Wo2 = Ho // 2, Wo // 2

    xn = jnp.transpose(x, (0, 2, 3, 1)).astype(jnp.bfloat16)
    xp = jnp.pad(xn, ((0, 0), (padding, padding), (padding, padding), (0, 0)))
    wmat = jnp.transpose(w, (2, 3, 1, 0)).astype(jnp.bfloat16)
    wmat = wmat.reshape(KH * KW * Cin, Cout)
    b2 = b.reshape(1, Cout).astype(jnp.float32)
    g2 = gamma.reshape(1, Cout).astype(jnp.float32)
    be2 = beta.reshape(1, Cout).astype(jnp.float32)
    Hp, Wp = xp.shape[1], xp.shape[2]

    k1 = functools.partial(_conv_pool_kernel, KH=KH, KW=KW)
    flops1 = 2 * N * Ho * Wo * KH * KW * Cin * Cout
    bytes1 = (2 * (xp.size + wmat.size) + 4 * b2.size
              + 2 * 2 * N * Ho2 * Wo2 * Cout + 4 * 2 * N * Cout)
    maxp, minp, stats = pl.pallas_call(
        k1,
        grid=(N,),
        in_specs=[
            pl.BlockSpec((1, Hp, Wp, Cin), lambda n: (n, 0, 0, 0)),
            pl.BlockSpec((KH * KW * Cin, Cout), lambda n: (0, 0)),
            pl.BlockSpec((1, Cout), lambda n: (0, 0)),
        ],
        out_specs=[
            pl.BlockSpec((1, Ho2, Wo2, Cout), lambda n: (n, 0, 0, 0)),
            pl.BlockSpec((1, Ho2, Wo2, Cout), lambda n: (n, 0, 0, 0)),
            pl.BlockSpec((1, 2, Cout), lambda n: (n, 0, 0)),
        ],
        out_shape=[
            jax.ShapeDtypeStruct((N, Ho2, Wo2, Cout), jnp.bfloat16),
            jax.ShapeDtypeStruct((N, Ho2, Wo2, Cout), jnp.bfloat16),
            jax.ShapeDtypeStruct((N, 2, Cout), jnp.float32),
        ],
        scratch_shapes=[pltpu.VMEM((Ho, Wo, Cout), jnp.float32)],
        compiler_params=pltpu.CompilerParams(dimension_semantics=("parallel",)),
        cost_estimate=pl.CostEstimate(flops=flops1, transcendentals=0,
                                      bytes_accessed=bytes1),
    )(xp, wmat, b2)

    count = N * Ho * Wo
    k2 = functools.partial(_bn_relu_kernel, count=count, eps=eps)
    flops2 = 8 * N * Ho2 * Wo2 * Cout
    bytes2 = (4 * stats.size + 4 * 2 * Cout
              + 2 * 2 * N * Ho2 * Wo2 * Cout + 4 * N * Ho2 * Wo2 * Cout)
    out = pl.pallas_call(
        k2,
        grid=(N,),
        in_specs=[
            pl.BlockSpec((N, 2, Cout), lambda n: (0, 0, 0)),
            pl.BlockSpec((1, Cout), lambda n: (0, 0)),
            pl.BlockSpec((1, Cout), lambda n: (0, 0)),
            pl.BlockSpec((1, Ho2, Wo2, Cout), lambda n: (n, 0, 0, 0)),
            pl.BlockSpec((1, Ho2, Wo2, Cout), lambda n: (n, 0, 0, 0)),
        ],
        out_specs=pl.BlockSpec((1, Ho2, Wo2, Cout), lambda n: (n, 0, 0, 0)),
        out_shape=jax.ShapeDtypeStruct((N, Ho2, Wo2, Cout), jnp.float32),
        compiler_params=pltpu.CompilerParams(dimension_semantics=("parallel",)),
        cost_estimate=pl.CostEstimate(flops=flops2, transcendentals=0,
                                      bytes_accessed=bytes2),
    )(stats, g2, be2, maxp, minp)

    return jnp.transpose(out, (0, 3, 1, 2))


# speedup vs baseline: 1.2678x; 1.2678x over previous
import functools

import jax
import jax.numpy as jnp
from jax.experimental import pallas as pl
from jax.experimental.pallas import tpu as pltpu


def _conv_pool_kernel(xp_ref, w_ref, b_ref, maxp_ref, minp_ref, stats_ref,
                      acc_ref, *, KH, KW):
    _, Ho2, Wo2, Cout = maxp_ref.shape
    Ho, Wo = 2 * Ho2, 2 * Wo2
    Cin = xp_ref.shape[3]
    rows = Ho * Wo

    cols = []
    for kh in range(KH):
        for kw in range(KW):
            cols.append(
                xp_ref[:, pl.ds(kh, Ho), pl.ds(kw, Wo), :].reshape(rows, Cin))
    patch = jnp.concatenate(cols, axis=1)

    acc = jnp.dot(patch, w_ref[...],
                  preferred_element_type=jnp.float32)
    acc = acc + b_ref[0]

    ssum = jnp.sum(acc, axis=0, keepdims=True)
    ssq = jnp.sum(acc * acc, axis=0, keepdims=True)
    stats_ref[...] = jnp.concatenate([ssum, ssq], axis=0).reshape(1, 2, Cout)

    acc_ref[...] = acc.reshape(Ho, Wo, Cout)
    mx = None
    mn = None
    for di in range(2):
        for dj in range(2):
            part = acc_ref[pl.ds(di, Ho2, 2), pl.ds(dj, Wo2, 2), :]
            mx = part if mx is None else jnp.maximum(mx, part)
            mn = part if mn is None else jnp.minimum(mn, part)
    maxp_ref[...] = mx.reshape(1, Ho2, Wo2, Cout).astype(maxp_ref.dtype)
    minp_ref[...] = mn.reshape(1, Ho2, Wo2, Cout).astype(minp_ref.dtype)


def _bn_relu_kernel(stats_ref, g_ref, be_ref, maxp_ref, minp_ref, o_ref,
                    *, count, eps):
    _, Ho2, Wo2, Cout = o_ref.shape

    ssum = jnp.sum(stats_ref[:, 0, :], axis=0, keepdims=True)
    ssq = jnp.sum(stats_ref[:, 1, :], axis=0, keepdims=True)
    mean = ssum / count
    var = ssq / count - mean * mean
    inv = jax.lax.rsqrt(var + eps)
    scale = g_ref[...] * inv
    shift = be_ref[...] - mean * scale

    shape = o_ref.shape
    scale4 = jnp.broadcast_to(scale.reshape(1, 1, 1, Cout), shape)
    shift4 = jnp.broadcast_to(shift.reshape(1, 1, 1, Cout), shape)
    sel = jnp.where(scale4 >= 0.0,
                    maxp_ref[...].astype(jnp.float32),
                    minp_ref[...].astype(jnp.float32))
    o_ref[...] = jnp.maximum(sel * scale4 + shift4, 0.0)


def kernel(x, w, b, gamma, beta):
    stride, padding, eps = 1, 1, 1e-5
    N, Cin, H, W = x.shape
    Cout, _, KH, KW = w.shape
    Ho = (H + 2 * padding - KH) // stride + 1
    Wo = (W + 2 * padding - KW) // stride + 1
    assert Ho % 2 == 0 and Wo % 2 == 0
    Ho2, Wo2 = Ho // 2, Wo // 2

    xn = jnp.transpose(x, (0, 2, 3, 1)).astype(jnp.float32)
    xp = jnp.pad(xn, ((0, 0), (padding, padding), (padding, padding), (0, 0)))
    wmat = jnp.transpose(w, (2, 3, 1, 0)).astype(jnp.float32)
    wmat = wmat.reshape(KH * KW * Cin, Cout)
    b2 = b.reshape(1, Cout).astype(jnp.float32)
    g2 = gamma.reshape(1, Cout).astype(jnp.float32)
    be2 = beta.reshape(1, Cout).astype(jnp.float32)
    Hp, Wp = xp.shape[1], xp.shape[2]

    k1 = functools.partial(_conv_pool_kernel, KH=KH, KW=KW)
    flops1 = 2 * N * Ho * Wo * KH * KW * Cin * Cout
    bytes1 = (4 * (xp.size + wmat.size) + 4 * b2.size
              + 2 * 2 * N * Ho2 * Wo2 * Cout + 4 * 2 * N * Cout)
    maxp, minp, stats = pl.pallas_call(
        k1,
        grid=(N,),
        in_specs=[
            pl.BlockSpec((1, Hp, Wp, Cin), lambda n: (n, 0, 0, 0)),
            pl.BlockSpec((KH * KW * Cin, Cout), lambda n: (0, 0)),
            pl.BlockSpec((1, Cout), lambda n: (0, 0)),
        ],
        out_specs=[
            pl.BlockSpec((1, Ho2, Wo2, Cout), lambda n: (n, 0, 0, 0)),
            pl.BlockSpec((1, Ho2, Wo2, Cout), lambda n: (n, 0, 0, 0)),
            pl.BlockSpec((1, 2, Cout), lambda n: (n, 0, 0)),
        ],
        out_shape=[
            jax.ShapeDtypeStruct((N, Ho2, Wo2, Cout), jnp.bfloat16),
            jax.ShapeDtypeStruct((N, Ho2, Wo2, Cout), jnp.bfloat16),
            jax.ShapeDtypeStruct((N, 2, Cout), jnp.float32),
        ],
        scratch_shapes=[pltpu.VMEM((Ho, Wo, Cout), jnp.float32)],
        compiler_params=pltpu.CompilerParams(dimension_semantics=("parallel",)),
        cost_estimate=pl.CostEstimate(flops=flops1, transcendentals=0,
                                      bytes_accessed=bytes1),
    )(xp, wmat, b2)

    count = N * Ho * Wo
    k2 = functools.partial(_bn_relu_kernel, count=count, eps=eps)
    flops2 = 8 * N * Ho2 * Wo2 * Cout
    bytes2 = (4 * stats.size + 4 * 2 * Cout
              + 2 * 2 * N * Ho2 * Wo2 * Cout + 4 * N * Ho2 * Wo2 * Cout)
    out = pl.pallas_call(
        k2,
        grid=(N,),
        in_specs=[
            pl.BlockSpec((N, 2, Cout), lambda n: (0, 0, 0)),
            pl.BlockSpec((1, Cout), lambda n: (0, 0)),
            pl.BlockSpec((1, Cout), lambda n: (0, 0)),
            pl.BlockSpec((1, Ho2, Wo2, Cout), lambda n: (n, 0, 0, 0)),
            pl.BlockSpec((1, Ho2, Wo2, Cout), lambda n: (n, 0, 0, 0)),
        ],
        out_specs=pl.BlockSpec((1, Ho2, Wo2, Cout), lambda n: (n, 0, 0, 0)),
        out_shape=jax.ShapeDtypeStruct((N, Ho2, Wo2, Cout), jnp.float32),
        compiler_params=pltpu.CompilerParams(dimension_semantics=("parallel",)),
        cost_estimate=pl.CostEstimate(flops=flops2, transcendentals=0,
                                      bytes_accessed=bytes2),
    )(stats, g2, be2, maxp, minp)

    return jnp.transpose(out, (0, 3, 1, 2))
